# baseline (speedup 1.0000x reference)
import functools

import jax
import jax.numpy as jnp
from jax import lax
from jax.experimental import pallas as pl
from jax.experimental.pallas import tpu as pltpu
from jax.experimental.pallas import tpu_sc as plsc

_B, _S, _V, _D, _T, _M = 4, 2048, 100000, 128, 2, 768
_LN_EPS = 1e-12
_IDX_CHUNK = 128


def _sc_gather(table, idx3):
  nw, n_chunks, _ = idx3.shape
  per_w = n_chunks * _IDX_CHUNK
  n = nw * per_w
  d = table.shape[1]
  info = plsc.get_sparse_core_info()
  mesh = plsc.VectorSubcoreMesh(core_axis_name="c", subcore_axis_name="s")

  @functools.partial(
      pl.kernel,
      mesh=mesh,
      out_type=jax.ShapeDtypeStruct((n, d), jnp.float32),
      scratch_types=[
          pltpu.VMEM((n_chunks, _IDX_CHUNK), jnp.int32),
          pltpu.VMEM((n_chunks, _IDX_CHUNK, d), jnp.float32),
          [pltpu.SemaphoreType.DMA] * n_chunks,
          pltpu.SemaphoreType.DMA,
      ],
  )
  def k(table_hbm, idx_hbm, out_hbm, idx_v, rows_v, gsems, ssem):
    wid = lax.axis_index("s") * info.num_cores + lax.axis_index("c")
    base = wid * per_w
    pltpu.sync_copy(idx_hbm.at[wid], idx_v)
    gathers = [
        pltpu.async_copy(table_hbm.at[idx_v.at[j]], rows_v.at[j], gsems[j])
        for j in range(n_chunks)
    ]
    stores = []
    for j in range(n_chunks):
      gathers[j].wait()
      stores.append(
          pltpu.async_copy(
              rows_v.at[j],
              out_hbm.at[pl.ds(base + j * _IDX_CHUNK, _IDX_CHUNK)], ssem))
    for s in stores:
      s.wait()

  return k(table, idx3)


def _dense_body(w_ref, pos_ref, tf_ref, tt_ref, ls_ref, lb_ref, W_ref, b_ref,
                o_ref):
  tf = tf_ref[...]
  tt0 = tt_ref[0:1, :]
  tt1 = tt_ref[1:2, :]
  blk = w_ref.shape[0]
  s = pos_ref.shape[0]
  w = w_ref[...]
  if blk > s:
    pos = (w.reshape(blk // s, s, _D) + pos_ref[...]).reshape(blk, _D)
  else:
    pos = w + pos_ref[...]
  x = pos + (tt0 + tf * (tt1 - tt0))
  mean = jnp.mean(x, axis=-1, keepdims=True)
  xc = x - mean
  var = jnp.mean(xc * xc, axis=-1, keepdims=True)
  normed = xc * lax.rsqrt(var + _LN_EPS)
  normed = normed * ls_ref[...] + lb_ref[...]
  o_ref[...] = (
      jnp.dot(normed.astype(jnp.bfloat16), W_ref[...].astype(jnp.bfloat16),
              preferred_element_type=jnp.float32)
      + b_ref[...])


def _tc_dense_chunk(wrows_c, pos2d, type_f_c, type_table, ln_scale, ln_bias,
                    W, b, blk, chunk_blk0, total_n, prev=None):
  n = wrows_c.shape[0]
  grid = (n // blk,)
  pos_blk = min(blk, _S)
  s_blocks = _S // pos_blk
  in_specs = [
      pl.BlockSpec((blk, _D), lambda i: (i, 0)),
      pl.BlockSpec((pos_blk, _D), lambda i: (i % s_blocks, 0)),
      pl.BlockSpec((blk, 1), lambda i: (i, 0)),
      pl.BlockSpec((_T, _D), lambda i: (0, 0)),
      pl.BlockSpec((1, _D), lambda i: (0, 0)),
      pl.BlockSpec((1, _D), lambda i: (0, 0)),
      pl.BlockSpec((_D, _M), lambda i: (0, 0)),
      pl.BlockSpec((1, _M), lambda i: (0, 0)),
  ]
  args = [wrows_c, pos2d, type_f_c, type_table, ln_scale, ln_bias, W, b]
  kwargs = {}
  body = _dense_body
  if prev is not None:
    in_specs.append(pl.BlockSpec(memory_space=pl.ANY))
    args.append(prev)
    kwargs["input_output_aliases"] = {8: 0}
    body = lambda *refs: _dense_body(*refs[:8], refs[-1])
  return pl.pallas_call(
      body,
      grid=grid,
      in_specs=in_specs,
      out_specs=pl.BlockSpec((blk, _M), lambda i: (i + chunk_blk0, 0)),
      out_shape=jax.ShapeDtypeStruct((total_n, _M), jnp.float32),
      compiler_params=pltpu.CompilerParams(
          dimension_semantics=("arbitrary",)),
      **kwargs,
  )(*args)


def kernel(input_ids, type_ids, word_table, pos_emb, type_table, ln_scale,
           ln_bias, W, b):
  bs = _B * _S
  nw = 32
  idx3 = input_ids.reshape(nw, (bs // nw) // _IDX_CHUNK, _IDX_CHUNK)
  wrows = _sc_gather(word_table, idx3)
  pos2d = pos_emb.reshape(_S, _D)
  type_f = type_ids.reshape(bs, 1).astype(jnp.float32)
  out = _tc_dense_chunk(wrows, pos2d, type_f, type_table,
                        ln_scale.reshape(1, _D), ln_bias.reshape(1, _D), W,
                        b.reshape(1, _M), blk=2048, chunk_blk0=0, total_n=bs)
  return out.reshape(_B, _S, _M)


# speedup vs baseline: 1.0285x; 1.0285x over previous
import functools

import jax
import jax.numpy as jnp
from jax import lax
from jax.experimental import pallas as pl
from jax.experimental.pallas import tpu as pltpu
from jax.experimental.pallas import tpu_sc as plsc

_B, _S, _V, _D, _T, _M = 4, 2048, 100000, 128, 2, 768
_LN_EPS = 1e-12
_IDX_CHUNK = 128


def _sc_gather(table, ids2d, chunk=64):
  bs_b, bs_s = ids2d.shape
  n = bs_b * bs_s
  d = table.shape[1]
  info = plsc.get_sparse_core_info()
  nw = info.num_cores * info.num_subcores
  per_w = n // nw
  w_per_row = bs_s // per_w
  n_chunks = per_w // chunk
  mesh = plsc.VectorSubcoreMesh(core_axis_name="c", subcore_axis_name="s")

  @functools.partial(
      pl.kernel,
      mesh=mesh,
      out_type=jax.ShapeDtypeStruct((n, d), jnp.float32),
      scratch_types=[
          pltpu.VMEM((per_w,), jnp.int32),
          pltpu.VMEM((n_chunks, chunk, d), jnp.float32),
          [pltpu.SemaphoreType.DMA] * n_chunks,
          pltpu.SemaphoreType.DMA,
      ],
  )
  def k(table_hbm, idx_hbm, out_hbm, idx_v, rows_v, gsems, ssem):
    wid = lax.axis_index("s") * info.num_cores + lax.axis_index("c")
    row = wid // w_per_row
    col = (wid % w_per_row) * per_w
    base = wid * per_w
    pltpu.sync_copy(idx_hbm.at[row, pl.ds(col, per_w)], idx_v)
    gathers = [
        pltpu.async_copy(table_hbm.at[idx_v.at[pl.ds(j * chunk, chunk)]],
                         rows_v.at[j], gsems[j])
        for j in range(n_chunks)
    ]
    stores = []
    for j in range(n_chunks):
      gathers[j].wait()
      stores.append(
          pltpu.async_copy(rows_v.at[j],
                           out_hbm.at[pl.ds(base + j * chunk, chunk)], ssem))
    for s in stores:
      s.wait()

  return k(table, ids2d)


def _dense_body(w_ref, pos_ref, tf_ref, tt_ref, ls_ref, lb_ref, W_ref, b_ref,
                o_ref):
  tf = tf_ref[...]
  tt0 = tt_ref[0:1, :]
  tt1 = tt_ref[1:2, :]
  blk = w_ref.shape[0]
  s = pos_ref.shape[0]
  w = w_ref[...]
  if blk > s:
    pos = (w.reshape(blk // s, s, _D) + pos_ref[...]).reshape(blk, _D)
  else:
    pos = w + pos_ref[...]
  x = pos + (tt0 + tf * (tt1 - tt0))
  mean = jnp.mean(x, axis=-1, keepdims=True)
  xc = x - mean
  var = jnp.mean(xc * xc, axis=-1, keepdims=True)
  normed = xc * lax.rsqrt(var + _LN_EPS)
  normed = normed * ls_ref[...] + lb_ref[...]
  o_ref[...] = (
      jnp.dot(normed.astype(jnp.bfloat16), W_ref[...].astype(jnp.bfloat16),
              preferred_element_type=jnp.float32)
      + b_ref[...])


def _tc_dense_chunk(wrows_c, pos2d, type_f_c, type_table, ln_scale, ln_bias,
                    W, b, blk, chunk_blk0, total_n, prev=None):
  n = wrows_c.shape[0]
  grid = (n // blk,)
  pos_blk = min(blk, _S)
  s_blocks = _S // pos_blk
  in_specs = [
      pl.BlockSpec((blk, _D), lambda i: (i, 0)),
      pl.BlockSpec((pos_blk, _D), lambda i: (i % s_blocks, 0)),
      pl.BlockSpec((blk, 1), lambda i: (i, 0)),
      pl.BlockSpec((_T, _D), lambda i: (0, 0)),
      pl.BlockSpec((1, _D), lambda i: (0, 0)),
      pl.BlockSpec((1, _D), lambda i: (0, 0)),
      pl.BlockSpec((_D, _M), lambda i: (0, 0)),
      pl.BlockSpec((1, _M), lambda i: (0, 0)),
  ]
  args = [wrows_c, pos2d, type_f_c, type_table, ln_scale, ln_bias, W, b]
  kwargs = {}
  body = _dense_body
  if prev is not None:
    in_specs.append(pl.BlockSpec(memory_space=pl.ANY))
    args.append(prev)
    kwargs["input_output_aliases"] = {8: 0}
    body = lambda *refs: _dense_body(*refs[:8], refs[-1])
  return pl.pallas_call(
      body,
      grid=grid,
      in_specs=in_specs,
      out_specs=pl.BlockSpec((blk, _M), lambda i: (i + chunk_blk0, 0)),
      out_shape=jax.ShapeDtypeStruct((total_n, _M), jnp.float32),
      compiler_params=pltpu.CompilerParams(
          dimension_semantics=("arbitrary",)),
      **kwargs,
  )(*args)


def kernel(input_ids, type_ids, word_table, pos_emb, type_table, ln_scale,
           ln_bias, W, b):
  bs = _B * _S
  wrows = _sc_gather(word_table, input_ids)
  pos2d = pos_emb.reshape(_S, _D)
  type_f = type_ids.reshape(bs, 1).astype(jnp.float32)
  out = _tc_dense_chunk(wrows, pos2d, type_f, type_table,
                        ln_scale.reshape(1, _D), ln_bias.reshape(1, _D), W,
                        b.reshape(1, _M), blk=2048, chunk_blk0=0, total_n=bs)
  return out.reshape(_B, _S, _M)
